# baseline (speedup 1.0000x reference)
import functools

import jax
import jax.numpy as jnp
from jax import lax
from jax.experimental import pallas as pl
from jax.experimental.pallas import tpu as pltpu
from jax.experimental.pallas import tpu_sc as plsc

_NB = 2048
_NW = 32
_NC = 2
_SMIN_INT = -2147483648
_I32_MAX = 0x7FFFFFFF
_N_CHUNKS_TC = 16


def _key16(v):
    return v ^ ((v >> 31) & jnp.int32(0x7FFFFFFF))


def _merge_scan(k, scan_bits, hp_hbm, si_hbm, mbuf_v, mrg_v, st_v):
    pltpu.sync_copy(si_hbm, st_v)
    st = st_v[...]
    iota = lax.iota(jnp.int32, 16)
    p_prev = jnp.sum(jnp.where(iota == 0, st, 0))
    ca_prev = jnp.sum(jnp.where(iota == 1, st, 0))

    pltpu.sync_copy(hp_hbm, mbuf_v)

    def mrow(j, _):
        def rrow(r, acc):
            return acc + mbuf_v[pl.ds(r * _NB + j * 16, 16)]

        mrg_v[pl.ds(j * 16, 16)] = lax.fori_loop(
            0, _NW, rrow, jnp.zeros((16,), jnp.int32))
        return 0

    lax.fori_loop(0, _NB // 16, mrow, 0)

    def tsum(j, acc):
        return acc + jnp.sum(mrg_v[pl.ds(j * 16, 16)])

    total = lax.fori_loop(0, _NB // 16, tsum, jnp.int32(0))
    m = total - (k - ca_prev)

    def scan(j, carry):
        e_run, b_acc, ca_acc = carry
        v = mrg_v[pl.ds(j * 16, 16)]
        cs = plsc.cumsum(v)
        incl = e_run + cs
        excl = incl - v
        hit = jnp.logical_and(excl <= m, m < incl)
        ids = j * 16 + lax.iota(jnp.int32, 16)
        b_acc = b_acc + jnp.sum(jnp.where(hit, ids, 0))
        ca_acc = ca_acc + jnp.sum(jnp.where(hit, total - incl, 0))
        return (e_run + jnp.sum(v), b_acc, ca_acc)

    _, b, ca_in = lax.fori_loop(
        0, _NB // 16, scan,
        (jnp.int32(0), jnp.int32(0), jnp.int32(0)))
    new_prefix = p_prev * jnp.int32(1 << scan_bits) + b
    return new_prefix, ca_prev + ca_in


def _phase_body(k, shard, n_chunks, chunk, scan_bits, chk_shift, bin_shift,
                nbits, x_hbm, hp_hbm, si_hbm, ho_hbm, so_hbm,
                buf_v, hist_v, mbuf_v, mrg_v, st_v):
    wid = lax.axis_index("s") * _NC + lax.axis_index("c")
    iota = lax.iota(jnp.int32, 16)

    if scan_bits is not None:
        prefix, ca = _merge_scan(k, scan_bits, hp_hbm, si_hbm,
                                 mbuf_v, mrg_v, st_v)
    else:
        prefix, ca = jnp.int32(0), jnp.int32(0)

    @pl.when(wid == 0)
    def _emit_state():
        st_v[...] = jnp.where(iota == 0, prefix,
                              jnp.where(iota == 1, ca, 0)).astype(jnp.int32)
        pltpu.sync_copy(st_v, so_hbm)

    def zero(j, _):
        hist_v[pl.ds(j * 16, 16)] = jnp.zeros((16,), jnp.int32)
        return 0

    lax.fori_loop(0, (16 * _NB) // 16, zero, 0)

    base = wid * shard
    lane = lax.iota(jnp.int32, 16)
    ones = jnp.ones((16,), jnp.int32)
    bmask = jnp.int32((1 << nbits) - 1)

    def do_chunk(ci, _):
        pltpu.sync_copy(x_hbm.at[pl.ds(base + ci * chunk, chunk)], buf_v)

        def inner(i, _):
            v = buf_v[pl.ds(i * 16, 16)]
            ku = _key16(v) ^ jnp.int32(_SMIN_INT)
            bn = lax.shift_right_logical(ku, bin_shift) & bmask
            idx = lane * _NB + bn
            if chk_shift is not None:
                msk = lax.shift_right_logical(ku, chk_shift) == prefix
                plsc.addupdate_scatter(hist_v, [idx], ones, mask=msk)
            else:
                plsc.addupdate_scatter(hist_v, [idx], ones)
            return 0

        lax.fori_loop(0, chunk // 16, inner, 0)
        return 0

    lax.fori_loop(0, n_chunks, do_chunk, 0)

    def col(j, _):
        def r16(r, acc):
            return acc + hist_v[pl.ds(r * _NB + j * 16, 16)]

        mrg_v[pl.ds(j * 16, 16)] = lax.fori_loop(
            0, 16, r16, jnp.zeros((16,), jnp.int32))
        return 0

    lax.fori_loop(0, _NB // 16, col, 0)
    pltpu.sync_copy(mrg_v, ho_hbm.at[pl.ds(wid * _NB, _NB)])


def _final_body(k, x_hbm, hp_hbm, si_hbm, so_hbm, mbuf_v, mrg_v, st_v):
    del x_hbm
    wid = lax.axis_index("s") * _NC + lax.axis_index("c")
    ku, c_gt = _merge_scan(k, 10, hp_hbm, si_hbm, mbuf_v, mrg_v, st_v)
    t_key = ku ^ jnp.int32(_SMIN_INT)

    @pl.when(wid == 0)
    def _emit():
        iota = lax.iota(jnp.int32, 16)
        st_v[...] = jnp.where(iota == 0, t_key,
                              jnp.where(iota == 1, c_gt, 0)).astype(jnp.int32)
        pltpu.sync_copy(st_v, so_hbm)


def _flat_ids(rows, d, c):
    row_ids = lax.broadcasted_iota(jnp.int32, (rows, d), 0) + c * rows
    lane_ids = lax.broadcasted_iota(jnp.int32, (rows, d), 1)
    return row_ids * d + lane_ids


def _tie_body(k, x_ref, st_ref, out_ref, s_ref):
    c = pl.program_id(0)
    rows, d = x_ref.shape
    t = st_ref[0]
    c_gt = st_ref[1]
    r = k - c_gt

    @pl.when(c == 0)
    def _init():
        s_ref[0] = jnp.int32(0)

    xi = lax.bitcast_convert_type(x_ref[...], jnp.int32)
    key = xi ^ ((xi >> 31) & jnp.int32(0x7FFFFFFF))
    eq = key == t
    c_block = jnp.sum(eq.astype(jnp.int32))
    flat = _flat_ids(rows, d, c)
    s_prev = s_ref[0]

    @pl.when(jnp.logical_and(s_prev < r, r <= s_prev + c_block))
    def _extract():
        need = r - s_prev

        def body(_, last):
            cand = jnp.where(jnp.logical_and(eq, flat > last), flat, _I32_MAX)
            return jnp.min(cand)

        cutoff = lax.fori_loop(0, need, body, jnp.int32(-1))
        out_ref[0] = t
        out_ref[1] = cutoff

    s_ref[0] = s_prev + c_block


def _mask_body(x_ref, tc_ref, o_ref):
    t = tc_ref[0]
    cut = tc_ref[1]
    rows, d = x_ref.shape
    xs = x_ref[...]
    xi = lax.bitcast_convert_type(xs, jnp.int32)
    key = xi ^ ((xi >> 31) & jnp.int32(0x7FFFFFFF))
    flat = _flat_ids(rows, d, pl.program_id(0))
    keep = jnp.logical_or(key > t, jnp.logical_and(key == t, flat <= cut))
    o_ref[...] = jnp.where(keep, xs, 0.0)


def _build_sc(n, k, interpret=False):
    shard = n // _NW
    chunk = min(8192, shard)
    n_chunks = shard // chunk
    mesh = plsc.VectorSubcoreMesh(core_axis_name="c", subcore_axis_name="s",
                                  num_cores=_NC, num_subcores=_NW // _NC)
    hist_t = jax.ShapeDtypeStruct((_NW * _NB,), jnp.int32)
    st_t = jax.ShapeDtypeStruct((16,), jnp.int32)
    scr = [
        pltpu.VMEM((chunk,), jnp.int32),
        pltpu.VMEM((16 * _NB,), jnp.int32),
        pltpu.VMEM((_NW * _NB,), jnp.int32),
        pltpu.VMEM((_NB,), jnp.int32),
        pltpu.VMEM((16,), jnp.int32),
    ]
    phases = []
    for scan_bits, chk_shift, bin_shift, nbits in (
            (None, None, 21, 11), (11, 21, 10, 11), (11, 10, 0, 10)):
        body = functools.partial(_phase_body, k, shard, n_chunks, chunk,
                                 scan_bits, chk_shift, bin_shift, nbits)
        phases.append(functools.partial(
            pl.kernel, mesh=mesh, out_type=[hist_t, st_t],
            scratch_types=scr, interpret=interpret,
            compiler_params=pltpu.CompilerParams(needs_layout_passes=False),
        )(body))
    fin = functools.partial(
        pl.kernel, mesh=mesh, out_type=[st_t],
        scratch_types=scr[2:], interpret=interpret,
        compiler_params=pltpu.CompilerParams(needs_layout_passes=False),
    )(functools.partial(_final_body, k))
    return phases, fin


def _build_tc(b, d, k, interpret=False):
    n_chunks = min(_N_CHUNKS_TC, b)
    rows = b // n_chunks
    tie = pl.pallas_call(
        functools.partial(_tie_body, k),
        grid=(n_chunks,),
        in_specs=[
            pl.BlockSpec((rows, d), lambda c: (c, 0)),
            pl.BlockSpec(memory_space=pltpu.SMEM),
        ],
        out_specs=pl.BlockSpec(memory_space=pltpu.SMEM),
        out_shape=jax.ShapeDtypeStruct((2,), jnp.int32),
        scratch_shapes=[pltpu.SMEM((1,), jnp.int32)],
        interpret=interpret,
    )
    mask = pl.pallas_call(
        _mask_body,
        grid=(n_chunks,),
        in_specs=[
            pl.BlockSpec((rows, d), lambda c: (c, 0)),
            pl.BlockSpec(memory_space=pltpu.SMEM),
        ],
        out_specs=pl.BlockSpec((rows, d), lambda c: (c, 0)),
        out_shape=jax.ShapeDtypeStruct((b, d), jnp.float32),
        interpret=interpret,
    )
    return tie, mask


def kernel(x):
    b, d = x.shape
    k = min(32 * b, b * d)
    n = b * d
    xf = lax.bitcast_convert_type(x, jnp.int32).reshape(-1)
    (ph_a, ph_b, ph_c), fin = _build_sc(n, k)
    z_hist = jnp.zeros((_NW * _NB,), jnp.int32)
    z_st = jnp.zeros((16,), jnp.int32)
    hist_a, _ = ph_a(xf, z_hist, z_st)
    hist_b, st_a = ph_b(xf, hist_a, z_st)
    hist_c, st_b = ph_c(xf, hist_b, st_a)
    (st_f,) = fin(xf, hist_c, st_b)
    tie, mask = _build_tc(b, d, k)
    tc = tie(x, st_f)
    return mask(x, tc)


# speedup vs baseline: 2.5821x; 2.5821x over previous
import functools

import jax
import jax.numpy as jnp
from jax import lax
from jax.experimental import pallas as pl
from jax.experimental.pallas import tpu as pltpu
from jax.experimental.pallas import tpu_sc as plsc

_NB = 2048
_NW = 32
_NC = 2
_SMIN_INT = -2147483648
_I32_MAX = 0x7FFFFFFF
_N_CHUNKS_TC = 16


def _key16(v):
    return v ^ ((v >> 31) & jnp.int32(0x7FFFFFFF))


def _merge_scan(k, scan_bits, hp_hbm, si_hbm, mbuf_v, mrg_v, st_v):
    pltpu.sync_copy(si_hbm, st_v)
    st = st_v[...]
    iota = lax.iota(jnp.int32, 16)
    p_prev = jnp.sum(jnp.where(iota == 0, st, 0))
    ca_prev = jnp.sum(jnp.where(iota == 1, st, 0))

    pltpu.sync_copy(hp_hbm, mbuf_v)

    @plsc.parallel_loop(0, _NB, 16, unroll=2)
    def mrow(j):
        def rrow(r, acc):
            return acc + mbuf_v[pl.ds(r * _NB + j, 16)]

        mrg_v[pl.ds(j, 16)] = lax.fori_loop(
            0, _NW, rrow, jnp.zeros((16,), jnp.int32))

    def tsum(j, acc):
        return acc + jnp.sum(mrg_v[pl.ds(j * 16, 16)])

    total = lax.fori_loop(0, _NB // 16, tsum, jnp.int32(0))
    m = total - (k - ca_prev)

    def scan(j, carry):
        e_run, b_acc, ca_acc = carry
        v = mrg_v[pl.ds(j * 16, 16)]
        cs = plsc.cumsum(v)
        incl = e_run + cs
        excl = incl - v
        hit = jnp.logical_and(excl <= m, m < incl)
        ids = j * 16 + lax.iota(jnp.int32, 16)
        b_acc = b_acc + jnp.sum(jnp.where(hit, ids, 0))
        ca_acc = ca_acc + jnp.sum(jnp.where(hit, total - incl, 0))
        return (e_run + jnp.sum(v), b_acc, ca_acc)

    _, b, ca_in = lax.fori_loop(
        0, _NB // 16, scan,
        (jnp.int32(0), jnp.int32(0), jnp.int32(0)))
    new_prefix = p_prev * jnp.int32(1 << scan_bits) + b
    return new_prefix, ca_prev + ca_in


def _phase_body(k, shard, n_chunks, chunk, scan_bits, chk_shift, bin_shift,
                nbits, x_hbm, hp_hbm, si_hbm, ho_hbm, so_hbm,
                buf_v, hist_v, mbuf_v, mrg_v, st_v):
    wid = lax.axis_index("s") * _NC + lax.axis_index("c")
    iota = lax.iota(jnp.int32, 16)

    if scan_bits is not None:
        prefix, ca = _merge_scan(k, scan_bits, hp_hbm, si_hbm,
                                 mbuf_v, mrg_v, st_v)
    else:
        prefix, ca = jnp.int32(0), jnp.int32(0)

    @pl.when(wid == 0)
    def _emit_state():
        st_v[...] = jnp.where(iota == 0, prefix,
                              jnp.where(iota == 1, ca, 0)).astype(jnp.int32)
        pltpu.sync_copy(st_v, so_hbm)

    @plsc.parallel_loop(0, 16 * _NB, 16, unroll=8)
    def zero(j):
        hist_v[pl.ds(j, 16)] = jnp.zeros((16,), jnp.int32)

    base = wid * shard
    lane = lax.iota(jnp.int32, 16)
    ones = jnp.ones((16,), jnp.int32)
    bmask = jnp.int32((1 << nbits) - 1)

    def do_chunk(ci, _):
        pltpu.sync_copy(x_hbm.at[pl.ds(base + ci * chunk, chunk)], buf_v)

        @plsc.parallel_loop(0, chunk, 16, unroll=8)
        def inner(i):
            v = buf_v[pl.ds(i, 16)]
            ku = _key16(v) ^ jnp.int32(_SMIN_INT)
            bn = lax.shift_right_logical(ku, bin_shift) & bmask
            idx = lane * _NB + bn
            if chk_shift is not None:
                msk = lax.shift_right_logical(ku, chk_shift) == prefix
                plsc.addupdate_scatter(hist_v, [idx], ones, mask=msk)
            else:
                plsc.addupdate_scatter(hist_v, [idx], ones)

        return 0

    lax.fori_loop(0, n_chunks, do_chunk, 0)

    @plsc.parallel_loop(0, _NB, 16, unroll=2)
    def col(j):
        def r16(r, acc):
            return acc + hist_v[pl.ds(r * _NB + j, 16)]

        mrg_v[pl.ds(j, 16)] = lax.fori_loop(
            0, 16, r16, jnp.zeros((16,), jnp.int32))
    pltpu.sync_copy(mrg_v, ho_hbm.at[pl.ds(wid * _NB, _NB)])


def _final_body(k, x_hbm, hp_hbm, si_hbm, so_hbm, mbuf_v, mrg_v, st_v):
    del x_hbm
    wid = lax.axis_index("s") * _NC + lax.axis_index("c")
    ku, c_gt = _merge_scan(k, 10, hp_hbm, si_hbm, mbuf_v, mrg_v, st_v)
    t_key = ku ^ jnp.int32(_SMIN_INT)

    @pl.when(wid == 0)
    def _emit():
        iota = lax.iota(jnp.int32, 16)
        st_v[...] = jnp.where(iota == 0, t_key,
                              jnp.where(iota == 1, c_gt, 0)).astype(jnp.int32)
        pltpu.sync_copy(st_v, so_hbm)


def _flat_ids(rows, d, c):
    row_ids = lax.broadcasted_iota(jnp.int32, (rows, d), 0) + c * rows
    lane_ids = lax.broadcasted_iota(jnp.int32, (rows, d), 1)
    return row_ids * d + lane_ids


def _tie_body(k, x_ref, st_ref, out_ref, s_ref):
    c = pl.program_id(0)
    rows, d = x_ref.shape
    t = st_ref[0]
    c_gt = st_ref[1]
    r = k - c_gt

    @pl.when(c == 0)
    def _init():
        s_ref[0] = jnp.int32(0)

    xi = lax.bitcast_convert_type(x_ref[...], jnp.int32)
    key = xi ^ ((xi >> 31) & jnp.int32(0x7FFFFFFF))
    eq = key == t
    c_block = jnp.sum(eq.astype(jnp.int32))
    flat = _flat_ids(rows, d, c)
    s_prev = s_ref[0]

    @pl.when(jnp.logical_and(s_prev < r, r <= s_prev + c_block))
    def _extract():
        need = r - s_prev

        def body(_, last):
            cand = jnp.where(jnp.logical_and(eq, flat > last), flat, _I32_MAX)
            return jnp.min(cand)

        cutoff = lax.fori_loop(0, need, body, jnp.int32(-1))
        out_ref[0] = t
        out_ref[1] = cutoff

    s_ref[0] = s_prev + c_block


def _mask_body(x_ref, tc_ref, o_ref):
    t = tc_ref[0]
    cut = tc_ref[1]
    rows, d = x_ref.shape
    xs = x_ref[...]
    xi = lax.bitcast_convert_type(xs, jnp.int32)
    key = xi ^ ((xi >> 31) & jnp.int32(0x7FFFFFFF))
    flat = _flat_ids(rows, d, pl.program_id(0))
    keep = jnp.logical_or(key > t, jnp.logical_and(key == t, flat <= cut))
    o_ref[...] = jnp.where(keep, xs, 0.0)


def _build_sc(n, k, interpret=False):
    shard = n // _NW
    chunk = min(8192, shard)
    n_chunks = shard // chunk
    mesh = plsc.VectorSubcoreMesh(core_axis_name="c", subcore_axis_name="s",
                                  num_cores=_NC, num_subcores=_NW // _NC)
    hist_t = jax.ShapeDtypeStruct((_NW * _NB,), jnp.int32)
    st_t = jax.ShapeDtypeStruct((16,), jnp.int32)
    scr = [
        pltpu.VMEM((chunk,), jnp.int32),
        pltpu.VMEM((16 * _NB,), jnp.int32),
        pltpu.VMEM((_NW * _NB,), jnp.int32),
        pltpu.VMEM((_NB,), jnp.int32),
        pltpu.VMEM((16,), jnp.int32),
    ]
    phases = []
    for scan_bits, chk_shift, bin_shift, nbits in (
            (None, None, 21, 11), (11, 21, 10, 11), (11, 10, 0, 10)):
        body = functools.partial(_phase_body, k, shard, n_chunks, chunk,
                                 scan_bits, chk_shift, bin_shift, nbits)
        phases.append(functools.partial(
            pl.kernel, mesh=mesh, out_type=[hist_t, st_t],
            scratch_types=scr, interpret=interpret,
            compiler_params=pltpu.CompilerParams(needs_layout_passes=False),
        )(body))
    fin = functools.partial(
        pl.kernel, mesh=mesh, out_type=[st_t],
        scratch_types=scr[2:], interpret=interpret,
        compiler_params=pltpu.CompilerParams(needs_layout_passes=False),
    )(functools.partial(_final_body, k))
    return phases, fin


def _build_tc(b, d, k, interpret=False):
    n_chunks = min(_N_CHUNKS_TC, b)
    rows = b // n_chunks
    tie = pl.pallas_call(
        functools.partial(_tie_body, k),
        grid=(n_chunks,),
        in_specs=[
            pl.BlockSpec((rows, d), lambda c: (c, 0)),
            pl.BlockSpec(memory_space=pltpu.SMEM),
        ],
        out_specs=pl.BlockSpec(memory_space=pltpu.SMEM),
        out_shape=jax.ShapeDtypeStruct((2,), jnp.int32),
        scratch_shapes=[pltpu.SMEM((1,), jnp.int32)],
        interpret=interpret,
    )
    mask = pl.pallas_call(
        _mask_body,
        grid=(n_chunks,),
        in_specs=[
            pl.BlockSpec((rows, d), lambda c: (c, 0)),
            pl.BlockSpec(memory_space=pltpu.SMEM),
        ],
        out_specs=pl.BlockSpec((rows, d), lambda c: (c, 0)),
        out_shape=jax.ShapeDtypeStruct((b, d), jnp.float32),
        interpret=interpret,
    )
    return tie, mask


def kernel(x):
    b, d = x.shape
    k = min(32 * b, b * d)
    n = b * d
    xf = lax.bitcast_convert_type(x, jnp.int32).reshape(-1)
    (ph_a, ph_b, ph_c), fin = _build_sc(n, k)
    z_hist = jnp.zeros((_NW * _NB,), jnp.int32)
    z_st = jnp.zeros((16,), jnp.int32)
    hist_a, _ = ph_a(xf, z_hist, z_st)
    hist_b, st_a = ph_b(xf, hist_a, z_st)
    hist_c, st_b = ph_c(xf, hist_b, st_a)
    (st_f,) = fin(xf, hist_c, st_b)
    tie, mask = _build_tc(b, d, k)
    tc = tie(x, st_f)
    return mask(x, tc)


# speedup vs baseline: 2.8261x; 1.0945x over previous
import functools

import jax
import jax.numpy as jnp
from jax import lax
from jax.experimental import pallas as pl
from jax.experimental.pallas import tpu as pltpu
from jax.experimental.pallas import tpu_sc as plsc

_NB = 2048
_NW = 32
_NC = 2
_SMIN_INT = -2147483648
_I32_MAX = 0x7FFFFFFF
_N_CHUNKS_TC = 16


def _key16(v):
    return v ^ ((v >> 31) & jnp.int32(0x7FFFFFFF))


def _merge_scan(k, scan_bits, hp_hbm, si_hbm, mbuf_v, mrg_v, st_v):
    pltpu.sync_copy(si_hbm, st_v)
    st = st_v[...]
    iota = lax.iota(jnp.int32, 16)
    p_prev = jnp.sum(jnp.where(iota == 0, st, 0))
    ca_prev = jnp.sum(jnp.where(iota == 1, st, 0))

    pltpu.sync_copy(hp_hbm, mbuf_v)

    @plsc.parallel_loop(0, _NB, 16, unroll=2)
    def mrow(j):
        def rrow(r, acc):
            return acc + mbuf_v[pl.ds(r * _NB + j, 16)]

        mrg_v[pl.ds(j, 16)] = lax.fori_loop(
            0, _NW, rrow, jnp.zeros((16,), jnp.int32))

    def tsum(j, acc):
        return acc + jnp.sum(mrg_v[pl.ds(j * 16, 16)])

    total = lax.fori_loop(0, _NB // 16, tsum, jnp.int32(0))
    m = total - (k - ca_prev)

    def scan(j, carry):
        e_run, b_acc, ca_acc = carry
        v = mrg_v[pl.ds(j * 16, 16)]
        cs = plsc.cumsum(v)
        incl = e_run + cs
        excl = incl - v
        hit = jnp.logical_and(excl <= m, m < incl)
        ids = j * 16 + lax.iota(jnp.int32, 16)
        b_acc = b_acc + jnp.sum(jnp.where(hit, ids, 0))
        ca_acc = ca_acc + jnp.sum(jnp.where(hit, total - incl, 0))
        return (e_run + jnp.sum(v), b_acc, ca_acc)

    _, b, ca_in = lax.fori_loop(
        0, _NB // 16, scan,
        (jnp.int32(0), jnp.int32(0), jnp.int32(0)))
    new_prefix = p_prev * jnp.int32(1 << scan_bits) + b
    return new_prefix, ca_prev + ca_in


def _phase_body(k, shard, n_chunks, chunk, scan_bits, chk_shift, bin_shift,
                nbits, x_hbm, hp_hbm, si_hbm, ho_hbm, so_hbm,
                buf_v, hist_v, mbuf_v, mrg_v, st_v):
    wid = lax.axis_index("s") * _NC + lax.axis_index("c")
    iota = lax.iota(jnp.int32, 16)

    if scan_bits is not None:
        prefix, ca = _merge_scan(k, scan_bits, hp_hbm, si_hbm,
                                 mbuf_v, mrg_v, st_v)
    else:
        prefix, ca = jnp.int32(0), jnp.int32(0)

    @pl.when(wid == 0)
    def _emit_state():
        st_v[...] = jnp.where(iota == 0, prefix,
                              jnp.where(iota == 1, ca, 0)).astype(jnp.int32)
        pltpu.sync_copy(st_v, so_hbm)

    @plsc.parallel_loop(0, 16 * _NB, 16, unroll=8)
    def zero(j):
        hist_v[pl.ds(j, 16)] = jnp.zeros((16,), jnp.int32)

    base = wid * shard
    lane = lax.iota(jnp.int32, 16)
    ones = jnp.ones((16,), jnp.int32)
    bmask = jnp.int32((1 << nbits) - 1)

    def do_chunk(ci, _):
        pltpu.sync_copy(x_hbm.at[pl.ds(base + ci * chunk, chunk)], buf_v)

        @plsc.parallel_loop(0, chunk, 16, unroll=8)
        def inner(i):
            v = plsc.bitcast(buf_v[pl.ds(i, 16)], jnp.int32)
            ku = _key16(v) ^ jnp.int32(_SMIN_INT)
            bn = lax.shift_right_logical(ku, bin_shift) & bmask
            idx = lane * _NB + bn
            if chk_shift is not None:
                msk = lax.shift_right_logical(ku, chk_shift) == prefix
                plsc.addupdate_scatter(hist_v, [idx], ones, mask=msk)
            else:
                plsc.addupdate_scatter(hist_v, [idx], ones)

        return 0

    lax.fori_loop(0, n_chunks, do_chunk, 0)

    @plsc.parallel_loop(0, _NB, 16, unroll=2)
    def col(j):
        def r16(r, acc):
            return acc + hist_v[pl.ds(r * _NB + j, 16)]

        mrg_v[pl.ds(j, 16)] = lax.fori_loop(
            0, 16, r16, jnp.zeros((16,), jnp.int32))
    pltpu.sync_copy(mrg_v, ho_hbm.at[pl.ds(wid * _NB, _NB)])


def _final_body(k, x_hbm, hp_hbm, si_hbm, so_hbm, mbuf_v, mrg_v, st_v):
    del x_hbm
    wid = lax.axis_index("s") * _NC + lax.axis_index("c")
    ku, c_gt = _merge_scan(k, 10, hp_hbm, si_hbm, mbuf_v, mrg_v, st_v)
    t_key = ku ^ jnp.int32(_SMIN_INT)

    @pl.when(wid == 0)
    def _emit():
        iota = lax.iota(jnp.int32, 16)
        st_v[...] = jnp.where(iota == 0, t_key,
                              jnp.where(iota == 1, c_gt, 0)).astype(jnp.int32)
        pltpu.sync_copy(st_v, so_hbm)


def _flat_ids(rows, d, c):
    row_ids = lax.broadcasted_iota(jnp.int32, (rows, d), 0) + c * rows
    lane_ids = lax.broadcasted_iota(jnp.int32, (rows, d), 1)
    return row_ids * d + lane_ids


def _mask_body(k, x_ref, st_ref, o_ref, s_ref):
    c = pl.program_id(0)
    rows, d = x_ref.shape
    t = st_ref[0]
    c_gt = st_ref[1]
    r = k - c_gt

    @pl.when(c == 0)
    def _init():
        s_ref[0] = jnp.int32(0)

    xs = x_ref[...]
    xi = lax.bitcast_convert_type(xs, jnp.int32)
    key = xi ^ ((xi >> 31) & jnp.int32(0x7FFFFFFF))
    eq = key == t
    c_block = jnp.sum(eq.astype(jnp.int32))
    flat = _flat_ids(rows, d, c)
    s_prev = s_ref[0]
    need = jnp.clip(r - s_prev, 0, c_block)

    def body(_, last):
        cand = jnp.where(jnp.logical_and(eq, flat > last), flat, _I32_MAX)
        return jnp.min(cand)

    full = need == c_block
    n_loop = jnp.where(full, 0, need)
    cut_loop = lax.fori_loop(0, n_loop, body, jnp.int32(-1))
    cut = jnp.where(full, jnp.max(jnp.where(eq, flat, -1)), cut_loop)
    keep = jnp.logical_or(key > t, jnp.logical_and(eq, flat <= cut))
    o_ref[...] = jnp.where(keep, xs, 0.0)
    s_ref[0] = s_prev + c_block


def _build_sc(n, k, interpret=False):
    shard = n // _NW
    chunk = min(8192, shard)
    n_chunks = shard // chunk
    mesh = plsc.VectorSubcoreMesh(core_axis_name="c", subcore_axis_name="s",
                                  num_cores=_NC, num_subcores=_NW // _NC)
    hist_t = jax.ShapeDtypeStruct((_NW * _NB,), jnp.int32)
    st_t = jax.ShapeDtypeStruct((16,), jnp.int32)
    scr = [
        pltpu.VMEM((chunk,), jnp.float32),
        pltpu.VMEM((16 * _NB,), jnp.int32),
        pltpu.VMEM((_NW * _NB,), jnp.int32),
        pltpu.VMEM((_NB,), jnp.int32),
        pltpu.VMEM((16,), jnp.int32),
    ]
    phases = []
    for scan_bits, chk_shift, bin_shift, nbits in (
            (None, None, 21, 11), (11, 21, 10, 11), (11, 10, 0, 10)):
        body = functools.partial(_phase_body, k, shard, n_chunks, chunk,
                                 scan_bits, chk_shift, bin_shift, nbits)
        phases.append(functools.partial(
            pl.kernel, mesh=mesh, out_type=[hist_t, st_t],
            scratch_types=scr, interpret=interpret,
            compiler_params=pltpu.CompilerParams(needs_layout_passes=False),
        )(body))
    fin = functools.partial(
        pl.kernel, mesh=mesh, out_type=[st_t],
        scratch_types=scr[2:], interpret=interpret,
        compiler_params=pltpu.CompilerParams(needs_layout_passes=False),
    )(functools.partial(_final_body, k))
    return phases, fin


def _build_tc(b, d, k, interpret=False):
    n_chunks = min(_N_CHUNKS_TC, b)
    rows = b // n_chunks
    mask = pl.pallas_call(
        functools.partial(_mask_body, k),
        grid=(n_chunks,),
        in_specs=[
            pl.BlockSpec((rows, d), lambda c: (c, 0)),
            pl.BlockSpec(memory_space=pltpu.SMEM),
        ],
        out_specs=pl.BlockSpec((rows, d), lambda c: (c, 0)),
        out_shape=jax.ShapeDtypeStruct((b, d), jnp.float32),
        scratch_shapes=[pltpu.SMEM((1,), jnp.int32)],
        interpret=interpret,
    )
    return mask


def kernel(x):
    b, d = x.shape
    k = min(32 * b, b * d)
    n = b * d
    xf = x.reshape(-1)
    (ph_a, ph_b, ph_c), fin = _build_sc(n, k)
    z_hist = jnp.zeros((_NW * _NB,), jnp.int32)
    z_st = jnp.zeros((16,), jnp.int32)
    hist_a, _ = ph_a(xf, z_hist, z_st)
    hist_b, st_a = ph_b(xf, hist_a, z_st)
    hist_c, st_b = ph_c(xf, hist_b, st_a)
    (st_f,) = fin(xf, hist_c, st_b)
    mask = _build_tc(b, d, k)
    return mask(x, st_f)


# speedup vs baseline: 3.1896x; 1.1286x over previous
import functools

import jax
import jax.numpy as jnp
from jax import lax
from jax.experimental import pallas as pl
from jax.experimental.pallas import tpu as pltpu
from jax.experimental.pallas import tpu_sc as plsc

_NB = 2048
_NW = 32
_NC = 2
_SMIN_INT = -2147483648
_I32_MAX = 0x7FFFFFFF
_N_CHUNKS_TC = 16


def _key16(v):
    return v ^ ((v >> 31) & jnp.int32(0x7FFFFFFF))


def _merge_scan(k, scan_bits, hp_hbm, si_hbm, mbuf_v, mrg_v, st_v):
    pltpu.sync_copy(si_hbm, st_v)
    st = st_v[...]
    iota = lax.iota(jnp.int32, 16)
    p_prev = jnp.sum(jnp.where(iota == 0, st, 0))
    ca_prev = jnp.sum(jnp.where(iota == 1, st, 0))

    pltpu.sync_copy(hp_hbm, mbuf_v)

    @plsc.parallel_loop(0, _NB, 16, unroll=2)
    def mrow(j):
        def rrow(r, acc):
            return acc + mbuf_v[pl.ds(r * _NB + j, 16)]

        mrg_v[pl.ds(j, 16)] = lax.fori_loop(
            0, _NW, rrow, jnp.zeros((16,), jnp.int32))

    def tsum(j, acc):
        return acc + jnp.sum(mrg_v[pl.ds(j * 16, 16)])

    total = lax.fori_loop(0, _NB // 16, tsum, jnp.int32(0))
    m = total - (k - ca_prev)

    def scan(j, carry):
        e_run, b_acc, ca_acc = carry
        v = mrg_v[pl.ds(j * 16, 16)]
        cs = plsc.cumsum(v)
        incl = e_run + cs
        excl = incl - v
        hit = jnp.logical_and(excl <= m, m < incl)
        ids = j * 16 + lax.iota(jnp.int32, 16)
        b_acc = b_acc + jnp.sum(jnp.where(hit, ids, 0))
        ca_acc = ca_acc + jnp.sum(jnp.where(hit, total - incl, 0))
        return (e_run + jnp.sum(v), b_acc, ca_acc)

    _, b, ca_in = lax.fori_loop(
        0, _NB // 16, scan,
        (jnp.int32(0), jnp.int32(0), jnp.int32(0)))
    new_prefix = p_prev * jnp.int32(1 << scan_bits) + b
    return new_prefix, ca_prev + ca_in


def _phase_body(k, shard, n_chunks, chunk, scan_bits, chk_shift, bin_shift,
                nbits, x_hbm, hp_hbm, si_hbm, ho_hbm, so_hbm,
                buf_v, hist_v, mbuf_v, mrg_v, st_v):
    wid = lax.axis_index("s") * _NC + lax.axis_index("c")
    iota = lax.iota(jnp.int32, 16)

    if scan_bits is not None:
        prefix, ca = _merge_scan(k, scan_bits, hp_hbm, si_hbm,
                                 mbuf_v, mrg_v, st_v)
    else:
        prefix, ca = jnp.int32(0), jnp.int32(0)

    @pl.when(wid == 0)
    def _emit_state():
        st_v[...] = jnp.where(iota == 0, prefix,
                              jnp.where(iota == 1, ca, 0)).astype(jnp.int32)
        pltpu.sync_copy(st_v, so_hbm)

    @plsc.parallel_loop(0, 16 * _NB, 16, unroll=8)
    def zero(j):
        hist_v[pl.ds(j, 16)] = jnp.zeros((16,), jnp.int32)

    base = wid * shard
    lane = lax.iota(jnp.int32, 16)
    ones = jnp.ones((16,), jnp.int32)
    bmask = jnp.int32((1 << nbits) - 1)

    def do_chunk(ci, _):
        pltpu.sync_copy(x_hbm.at[pl.ds(base + ci * chunk, chunk)], buf_v)

        @plsc.parallel_loop(0, chunk, 16, unroll=8)
        def inner(i):
            v = plsc.bitcast(buf_v[pl.ds(i, 16)], jnp.int32)
            ku = _key16(v) ^ jnp.int32(_SMIN_INT)
            bn = lax.shift_right_logical(ku, bin_shift) & bmask
            idx = lane * _NB + bn
            if chk_shift is not None:
                msk = lax.shift_right_logical(ku, chk_shift) == prefix
                plsc.addupdate_scatter(hist_v, [idx], ones, mask=msk)
            else:
                plsc.addupdate_scatter(hist_v, [idx], ones)

        return 0

    lax.fori_loop(0, n_chunks, do_chunk, 0)

    @plsc.parallel_loop(0, _NB, 16, unroll=2)
    def col(j):
        def r16(r, acc):
            return acc + hist_v[pl.ds(r * _NB + j, 16)]

        mrg_v[pl.ds(j, 16)] = lax.fori_loop(
            0, 16, r16, jnp.zeros((16,), jnp.int32))
    pltpu.sync_copy(mrg_v, ho_hbm.at[pl.ds(wid * _NB, _NB)])


def _final_body(k, x_hbm, hp_hbm, si_hbm, so_hbm, mbuf_v, mrg_v, st_v):
    del x_hbm
    wid = lax.axis_index("s") * _NC + lax.axis_index("c")
    ku, c_gt = _merge_scan(k, 10, hp_hbm, si_hbm, mbuf_v, mrg_v, st_v)
    t_key = ku ^ jnp.int32(_SMIN_INT)

    @pl.when(wid == 0)
    def _emit():
        iota = lax.iota(jnp.int32, 16)
        st_v[...] = jnp.where(iota == 0, t_key,
                              jnp.where(iota == 1, c_gt, 0)).astype(jnp.int32)
        pltpu.sync_copy(st_v, so_hbm)


def _flat_ids(rows, d, c):
    row_ids = lax.broadcasted_iota(jnp.int32, (rows, d), 0) + c * rows
    lane_ids = lax.broadcasted_iota(jnp.int32, (rows, d), 1)
    return row_ids * d + lane_ids


def _mask_body(k, x_ref, st_ref, o_ref, s_ref):
    c = pl.program_id(0)
    rows, d = x_ref.shape
    t = st_ref[0]
    c_gt = st_ref[1]
    r = k - c_gt

    @pl.when(c == 0)
    def _init():
        s_ref[0] = jnp.int32(0)

    xs = x_ref[...]
    xi = lax.bitcast_convert_type(xs, jnp.int32)
    key = xi ^ ((xi >> 31) & jnp.int32(0x7FFFFFFF))
    eq = key == t
    c_block = jnp.sum(eq.astype(jnp.int32))
    flat = _flat_ids(rows, d, c)
    s_prev = s_ref[0]
    need = jnp.clip(r - s_prev, 0, c_block)

    def body(_, last):
        cand = jnp.where(jnp.logical_and(eq, flat > last), flat, _I32_MAX)
        return jnp.min(cand)

    full = need == c_block
    n_loop = jnp.where(full, 0, need)
    cut_loop = lax.fori_loop(0, n_loop, body, jnp.int32(-1))
    cut = jnp.where(full, jnp.max(jnp.where(eq, flat, -1)), cut_loop)
    keep = jnp.logical_or(key > t, jnp.logical_and(eq, flat <= cut))
    o_ref[...] = jnp.where(keep, xs, 0.0)
    s_ref[0] = s_prev + c_block


def _build_sc(n, k, interpret=False):
    shard = n // _NW
    chunk = min(16384, shard)
    n_chunks = shard // chunk
    mesh = plsc.VectorSubcoreMesh(core_axis_name="c", subcore_axis_name="s",
                                  num_cores=_NC, num_subcores=_NW // _NC)
    hist_t = jax.ShapeDtypeStruct((_NW * _NB,), jnp.int32)
    st_t = jax.ShapeDtypeStruct((16,), jnp.int32)
    scr = [
        pltpu.VMEM((chunk,), jnp.float32),
        pltpu.VMEM((16 * _NB,), jnp.int32),
        pltpu.VMEM((_NW * _NB,), jnp.int32),
        pltpu.VMEM((_NB,), jnp.int32),
        pltpu.VMEM((16,), jnp.int32),
    ]
    phases = []
    for scan_bits, chk_shift, bin_shift, nbits in (
            (None, None, 21, 11), (11, 21, 10, 11), (11, 10, 0, 10)):
        body = functools.partial(_phase_body, k, shard, n_chunks, chunk,
                                 scan_bits, chk_shift, bin_shift, nbits)
        phases.append(functools.partial(
            pl.kernel, mesh=mesh, out_type=[hist_t, st_t],
            scratch_types=scr, interpret=interpret,
            compiler_params=pltpu.CompilerParams(needs_layout_passes=False),
        )(body))
    fin = functools.partial(
        pl.kernel, mesh=mesh, out_type=[st_t],
        scratch_types=scr[2:], interpret=interpret,
        compiler_params=pltpu.CompilerParams(needs_layout_passes=False),
    )(functools.partial(_final_body, k))
    return phases, fin


def _build_tc(b, d, k, interpret=False):
    n_chunks = min(_N_CHUNKS_TC, b)
    rows = b // n_chunks
    mask = pl.pallas_call(
        functools.partial(_mask_body, k),
        grid=(n_chunks,),
        in_specs=[
            pl.BlockSpec((rows, d), lambda c: (c, 0)),
            pl.BlockSpec(memory_space=pltpu.SMEM),
        ],
        out_specs=pl.BlockSpec((rows, d), lambda c: (c, 0)),
        out_shape=jax.ShapeDtypeStruct((b, d), jnp.float32),
        scratch_shapes=[pltpu.SMEM((1,), jnp.int32)],
        interpret=interpret,
    )
    return mask


def kernel(x):
    b, d = x.shape
    k = min(32 * b, b * d)
    n = b * d
    xf = x.reshape(-1)
    (ph_a, ph_b, ph_c), fin = _build_sc(n, k)
    z_hist = jnp.zeros((_NW * _NB,), jnp.int32)
    z_st = jnp.zeros((16,), jnp.int32)
    hist_a, _ = ph_a(xf, z_hist, z_st)
    hist_b, st_a = ph_b(xf, hist_a, z_st)
    hist_c, st_b = ph_c(xf, hist_b, st_a)
    (st_f,) = fin(xf, hist_c, st_b)
    mask = _build_tc(b, d, k)
    return mask(x, st_f)
